# baseline (speedup 1.0000x reference)
import jax
import jax.numpy as jnp
import numpy as np
from jax.experimental import pallas as pl

V = 2048
B = 4
BIG = 1e30


def _chamfer_body(xe_ref, ye_ref, out_ref):
    t = pl.program_id(0)
    x = xe_ref[0, 0]
    y = ye_ref[0, 0]
    d = jax.lax.dot_general(
        x, y, (((1,), (1,)), ((), ())), preferred_element_type=jnp.float32
    )
    count = (V - t).astype(jnp.float32)
    rowmin = jnp.min(d, axis=1, keepdims=True)
    colmin = jnp.min(d, axis=0, keepdims=True)
    rio = jax.lax.broadcasted_iota(jnp.int32, (V, 1), 0)
    cio = jax.lax.broadcasted_iota(jnp.int32, (1, V), 1)
    rowsum = jnp.sum(jnp.where(rio < V - t, rowmin, 0.0))
    colsum = jnp.sum(jnp.where(cio < V - t, colmin, 0.0))
    out_ref[...] = ((rowsum + colsum) / count).reshape(1, 1, 1, 1)


def _embed_queries(p):
    n = p.shape[1]
    x2 = jnp.sum(p * p, axis=-1, keepdims=True)
    one = jnp.ones_like(x2)
    zero = jnp.zeros((p.shape[0], n, 3), p.dtype)
    return jnp.concatenate([-2.0 * p, x2, one, zero], axis=-1)


def _embed_keys(tgt):
    n = tgt.shape[1]
    y2 = jnp.sum(tgt * tgt, axis=-1, keepdims=True)
    one = jnp.ones_like(y2)
    zero = jnp.zeros((tgt.shape[0], n, 3), tgt.dtype)
    return jnp.concatenate([tgt, one, y2, zero], axis=-1)


def _pad_row(e, row):
    pad = jnp.broadcast_to(jnp.asarray(row, e.dtype), (e.shape[0], 1, 8))
    return jnp.concatenate([e, pad], axis=1)


def _chamfer_pallas(predictions, targets):
    xe_full = _embed_queries(predictions)
    ye_full = _embed_keys(targets)
    ps = predictions[:, 1:, :] - predictions[:, :-1, :]
    ts = targets[:, 1:, :] - targets[:, :-1, :]
    xe_s = _pad_row(_embed_queries(ps), [0, 0, 0, BIG, 1, 0, 0, 0])
    ye_s = _pad_row(_embed_keys(ts), [0, 0, 0, 1, BIG, 0, 0, 0])
    xe = jnp.stack([xe_full, xe_s])
    ye = jnp.stack([ye_full, ye_s])
    out = pl.pallas_call(
        _chamfer_body,
        grid=(2, B),
        in_specs=[
            pl.BlockSpec((1, 1, V, 8), lambda t, b: (t, b, 0, 0)),
            pl.BlockSpec((1, 1, V, 8), lambda t, b: (t, b, 0, 0)),
        ],
        out_specs=pl.BlockSpec((1, 1, 1, 1), lambda t, b: (t, b, 0, 0)),
        out_shape=jax.ShapeDtypeStruct((2, B, 1, 1), jnp.float32),
    )(xe, ye)
    return jnp.mean(out[0]), jnp.mean(out[1])


def _sn(x):
    return jnp.sqrt(jnp.sum(x * x, axis=1) + 1e-20)


def _edge_loss(verts, faces):
    Bn, Vn, _ = verts.shape
    F = faces.shape[1]
    total = 0.0
    for b in range(Bn):
        f = faces[b]
        e = jnp.concatenate([f[:, [0, 1]], f[:, [1, 2]], f[:, [2, 0]]], axis=0)
        e = jnp.sort(e, axis=1)
        keys = e[:, 0] * Vn + e[:, 1]
        uniq = jnp.unique(keys, size=3 * F, fill_value=-1)
        validb = uniq >= 0
        valid = validb.astype(verts.dtype)
        uq = jnp.where(validb, uniq, 0)
        v0 = verts[b][uq // Vn]
        v1 = verts[b][uq % Vn]
        l2 = jnp.sum((v0 - v1) ** 2, axis=1) * valid
        num = jnp.maximum(jnp.sum(valid), 1.0)
        total = total + jnp.sum(l2) / num
    return total / Bn


def _laplacian_loss(verts, faces):
    Bn, Vn, _ = verts.shape
    total = 0.0
    for b in range(Bn):
        vs = verts[b]
        f = faces[b]
        v0, v1, v2 = vs[f[:, 0]], vs[f[:, 1]], vs[f[:, 2]]
        A = _sn(v1 - v2)
        Bl = _sn(v0 - v2)
        C = _sn(v0 - v1)
        s = 0.5 * (A + Bl + C)
        area = jnp.sqrt(jnp.clip(s * (s - A) * (s - Bl) * (s - C), 1e-12, None))
        A2, B2, C2 = A * A, Bl * Bl, C * C
        cota = (B2 + C2 - A2) / area
        cotb = (A2 + C2 - B2) / area
        cotc = (A2 + B2 - C2) / area
        cot = (jnp.stack([cota, cotb, cotc], axis=1) / 4.0).reshape(-1)
        ii = f[:, [1, 2, 0]].reshape(-1)
        jj = f[:, [2, 0, 1]].reshape(-1)
        Lx = jnp.zeros((Vn, 3), dtype=verts.dtype)
        Lx = Lx.at[ii].add(cot[:, None] * vs[jj])
        Lx = Lx.at[jj].add(cot[:, None] * vs[ii])
        rowsum = jnp.zeros((Vn,), dtype=verts.dtype)
        rowsum = rowsum.at[ii].add(cot)
        rowsum = rowsum.at[jj].add(cot)
        safe = jnp.where(rowsum > 0, rowsum, 1.0)
        norm_w = jnp.where(rowsum > 0, 1.0 / safe, 0.0)[:, None]
        diff = Lx * norm_w - vs
        total = total + jnp.sum(_sn(diff)) / Vn
    return total / Bn


def _normal_loss(verts, faces):
    Bn, Vn, _ = verts.shape
    total = 0.0
    for b in range(Bn):
        vs = verts[b]
        f = faces[b]
        e = jnp.concatenate([f[:, [0, 1]], f[:, [1, 2]], f[:, [2, 0]]], axis=0)
        opp = jnp.concatenate([f[:, 2], f[:, 0], f[:, 1]], axis=0)
        e = jnp.sort(e, axis=1)
        keys = e[:, 0] * Vn + e[:, 1]
        order = jnp.argsort(keys)
        ks = keys[order]
        opps = opp[order]
        mask = (ks[1:] == ks[:-1]).astype(verts.dtype)
        k0 = ks[:-1]
        ev0 = vs[k0 // Vn]
        ev1 = vs[k0 % Vn]
        va = vs[opps[:-1]]
        vb = vs[opps[1:]]
        n0 = jnp.cross(va - ev1, ev0 - ev1)
        n1 = -jnp.cross(vb - ev1, ev0 - ev1)
        n0m = jnp.maximum(_sn(n0), 1e-8)
        n1m = jnp.maximum(_sn(n1), 1e-8)
        cos = jnp.sum(n0 * n1, axis=1) / (n0m * n1m)
        pair_loss = (1.0 - cos) * mask
        num = jnp.sum(mask)
        total = total + jnp.where(num > 0, jnp.sum(pair_loss) / jnp.maximum(num, 1.0), 0.0)
    return total / Bn


def kernel(predictions, targets, pred_faces):
    loss_chamfer, vel_loss = _chamfer_pallas(predictions, targets)
    w_edge = 0.5 * _edge_loss(predictions, pred_faces)
    w_lap = 0.05 * _laplacian_loss(predictions, pred_faces)
    w_norm = 0.01 * _normal_loss(predictions, pred_faces)
    return loss_chamfer + w_lap + w_norm + w_edge + 10.0 * vel_loss


# speedup vs baseline: 20.7847x; 20.7847x over previous
import functools

import jax
import jax.numpy as jnp
from jax import lax
from jax.experimental import pallas as pl
from jax.experimental.pallas import tpu as pltpu, tpu_sc as plsc

V = 2048
B = 4
F = 4096
E = 3 * F
BIG = 1e30


def _chamfer_body(xe_ref, ye_ref, out_ref):
    t = pl.program_id(0)
    x = xe_ref[0, 0]
    y = ye_ref[0, 0]
    d = jax.lax.dot_general(
        x, y, (((1,), (1,)), ((), ())), preferred_element_type=jnp.float32
    )
    count = (V - t).astype(jnp.float32)
    rowmin = jnp.min(d, axis=1, keepdims=True)
    colmin = jnp.min(d, axis=0, keepdims=True)
    rio = jax.lax.broadcasted_iota(jnp.int32, (V, 1), 0)
    cio = jax.lax.broadcasted_iota(jnp.int32, (1, V), 1)
    rowsum = jnp.sum(jnp.where(rio < V - t, rowmin, 0.0))
    colsum = jnp.sum(jnp.where(cio < V - t, colmin, 0.0))
    out_ref[...] = ((rowsum + colsum) / count).reshape(1, 1, 1, 1)


def _embed_queries(p):
    n = p.shape[1]
    x2 = jnp.sum(p * p, axis=-1, keepdims=True)
    one = jnp.ones_like(x2)
    zero = jnp.zeros((p.shape[0], n, 3), p.dtype)
    return jnp.concatenate([-2.0 * p, x2, one, zero], axis=-1)


def _embed_keys(tgt):
    n = tgt.shape[1]
    y2 = jnp.sum(tgt * tgt, axis=-1, keepdims=True)
    one = jnp.ones_like(y2)
    zero = jnp.zeros((tgt.shape[0], n, 3), tgt.dtype)
    return jnp.concatenate([tgt, one, y2, zero], axis=-1)


def _pad_row(e, row):
    pad = jnp.broadcast_to(jnp.asarray(row, e.dtype), (e.shape[0], 1, 8))
    return jnp.concatenate([e, pad], axis=1)


def _chamfer_pallas(predictions, targets):
    xe_full = _embed_queries(predictions)
    ye_full = _embed_keys(targets)
    ps = predictions[:, 1:, :] - predictions[:, :-1, :]
    ts = targets[:, 1:, :] - targets[:, :-1, :]
    xe_s = _pad_row(_embed_queries(ps), [0, 0, 0, BIG, 1, 0, 0, 0])
    ye_s = _pad_row(_embed_keys(ts), [0, 0, 0, 1, BIG, 0, 0, 0])
    xe = jnp.stack([xe_full, xe_s])
    ye = jnp.stack([ye_full, ye_s])
    out = pl.pallas_call(
        _chamfer_body,
        grid=(2, B),
        in_specs=[
            pl.BlockSpec((1, 1, V, 8), lambda t, b: (t, b, 0, 0)),
            pl.BlockSpec((1, 1, V, 8), lambda t, b: (t, b, 0, 0)),
        ],
        out_specs=pl.BlockSpec((1, 1, 1, 1), lambda t, b: (t, b, 0, 0)),
        out_shape=jax.ShapeDtypeStruct((2, B, 1, 1), jnp.float32),
    )(xe, ye)
    return jnp.mean(out[0]), jnp.mean(out[1])


NW = 8
FW = F // NW
VW = V // NW


def _sqrtv(x):
    i = plsc.bitcast(x, jnp.int32)
    y = plsc.bitcast(0x5F3759DF - (i >> 1), jnp.float32)
    y = y * (1.5 - 0.5 * x * y * y)
    y = y * (1.5 - 0.5 * x * y * y)
    y = y * (1.5 - 0.5 * x * y * y)
    return x * y


def _sc_mesh_body(vf, ff, out, vx, vy, vz, fa, fb, fc, ea, pk, eo, srt,
                  off, end_, cur, lx, ly, lz, lw, tm, mlx, mly, mlz, mlw,
                  stg, sh_ea, sh_pk, sh_eo, sh_srt, sh_off, sh_end, sh_lap,
                  phases=9):
    c = lax.axis_index("c")
    s = lax.axis_index("s")
    bic = s >> 3
    sub = s & 7
    batch = c * 2 + bic
    iot = lax.iota(jnp.int32, 16)
    zf = jnp.zeros((16,), jnp.float32)
    zi = jnp.zeros((16,), jnp.int32)

    vbase = batch * (3 * V)
    pltpu.sync_copy(vf.at[pl.ds(vbase, V)], vx)
    pltpu.sync_copy(vf.at[pl.ds(vbase + V, V)], vy)
    pltpu.sync_copy(vf.at[pl.ds(vbase + 2 * V, V)], vz)
    fbase = batch * (3 * F)
    pltpu.sync_copy(ff.at[pl.ds(fbase, F)], fa)
    pltpu.sync_copy(ff.at[pl.ds(fbase + F, F)], fb)
    pltpu.sync_copy(ff.at[pl.ds(fbase + 2 * F, F)], fc)

    def zero4(i, _):
        lx[pl.ds(i * 16, 16)] = zf
        ly[pl.ds(i * 16, 16)] = zf
        lz[pl.ds(i * 16, 16)] = zf
        lw[pl.ds(i * 16, 16)] = zf
        return 0
    lax.fori_loop(0, V // 16, zero4, 0)

    fstart = sub * FW

    def p1(i, _):
        base = fstart + i * 16
        i0 = fa[pl.ds(base, 16)]
        i1 = fb[pl.ds(base, 16)]
        i2 = fc[pl.ds(base, 16)]
        v0x = plsc.load_gather(vx, [i0])
        v0y = plsc.load_gather(vy, [i0])
        v0z = plsc.load_gather(vz, [i0])
        v1x = plsc.load_gather(vx, [i1])
        v1y = plsc.load_gather(vy, [i1])
        v1z = plsc.load_gather(vz, [i1])
        v2x = plsc.load_gather(vx, [i2])
        v2y = plsc.load_gather(vy, [i2])
        v2z = plsc.load_gather(vz, [i2])
        def d2(px, py, pz, qx, qy, qz):
            ex, ey, ez = px - qx, py - qy, pz - qz
            return ex * ex + ey * ey + ez * ez
        a2 = d2(v1x, v1y, v1z, v2x, v2y, v2z) + 1e-20
        b2 = d2(v0x, v0y, v0z, v2x, v2y, v2z) + 1e-20
        c2 = d2(v0x, v0y, v0z, v1x, v1y, v1z) + 1e-20
        an = _sqrtv(a2)
        bn = _sqrtv(b2)
        cn = _sqrtv(c2)
        sp = 0.5 * (an + bn + cn)
        h = jnp.maximum(sp * (sp - an) * (sp - bn) * (sp - cn), 1e-12)
        inv4 = 0.25 / _sqrtv(h)
        cota = (b2 + c2 - a2) * inv4
        cotb = (a2 + c2 - b2) * inv4
        cotc = (a2 + b2 - c2) * inv4
        for ct, ii, jj, vix, viy, viz, vjx, vjy, vjz in (
            (cota, i1, i2, v1x, v1y, v1z, v2x, v2y, v2z),
            (cotb, i2, i0, v2x, v2y, v2z, v0x, v0y, v0z),
            (cotc, i0, i1, v0x, v0y, v0z, v1x, v1y, v1z),
        ):
            plsc.addupdate_scatter(lx, [ii], ct * vjx)
            plsc.addupdate_scatter(ly, [ii], ct * vjy)
            plsc.addupdate_scatter(lz, [ii], ct * vjz)
            plsc.addupdate_scatter(lw, [ii], ct)
            plsc.addupdate_scatter(lx, [jj], ct * vix)
            plsc.addupdate_scatter(ly, [jj], ct * viy)
            plsc.addupdate_scatter(lz, [jj], ct * viz)
            plsc.addupdate_scatter(lw, [jj], ct)
        for blk, (u, w, o) in enumerate(((i0, i1, i2), (i1, i2, i0), (i2, i0, i1))):
            a_ = jnp.minimum(u, w)
            b_ = jnp.maximum(u, w)
            t0 = blk * F + base
            ea[pl.ds(t0, 16)] = a_
            pk[pl.ds(t0, 16)] = (b_ << 14) + t0 + iot
            eo[pl.ds(t0, 16)] = o
        return 0
    if phases >= 1:
        lax.fori_loop(0, FW // 16, p1, 0)

    if phases >= 2:
        for blk in range(3):
            t0c = blk * F + fstart
            pltpu.sync_copy(ea.at[pl.ds(t0c, FW)], sh_ea.at[pl.ds(bic * E + t0c, FW)])
            pltpu.sync_copy(pk.at[pl.ds(t0c, FW)], sh_pk.at[pl.ds(bic * E + t0c, FW)])
            pltpu.sync_copy(eo.at[pl.ds(t0c, FW)], sh_eo.at[pl.ds(bic * E + t0c, FW)])
        for qi, r in enumerate((lx, ly, lz, lw)):
            pltpu.sync_copy(r, sh_lap.at[pl.ds((((bic * 8 + sub) * 4) + qi) * V, V)])
        plsc.subcore_barrier()

    @pl.when((sub == 0) & (phases >= 3))
    def _():
        pltpu.sync_copy(sh_ea.at[pl.ds(bic * E, E)], ea)
        pltpu.sync_copy(sh_pk.at[pl.ds(bic * E, E)], pk)

        def zc(i, _):
            cur[pl.ds(i * 16, 16)] = zi
            return 0
        lax.fori_loop(0, V // 16, zc, 0)

        def hist(i, _):
            a_ = ea[pl.ds(i * 16, 16)]
            plsc.addupdate_scatter(cur, [a_], jnp.ones((16,), jnp.int32))
            return 0
        lax.fori_loop(0, E // 16, hist, 0)

        def pfx(i, carry):
            v = cur[pl.ds(i * 16, 16)]
            cs = plsc.cumsum(v)
            offv = cs - v + carry
            off[pl.ds(i * 16, 16)] = offv
            end_[pl.ds(i * 16, 16)] = cs + carry
            cur[pl.ds(i * 16, 16)] = offv
            return carry + jnp.max(cs)
        lax.fori_loop(0, V // 16, pfx, jnp.int32(0))

        def scat(i, _):
            a_ = ea[pl.ds(i * 16, 16)]
            p_ = pk[pl.ds(i * 16, 16)]
            basev = plsc.load_gather(cur, [a_])
            rc = plsc.scan_count(a_)
            slot = basev + rc[0] - 1
            plsc.store_scatter(srt, [slot], p_)
            plsc.addupdate_scatter(cur, [a_], rc[0] * rc[1])
            return 0
        lax.fori_loop(0, E // 16, scat, 0)
        pltpu.sync_copy(srt, sh_srt.at[pl.ds(bic * E, E)])
        pltpu.sync_copy(off, sh_off.at[pl.ds(bic * V, V)])
        pltpu.sync_copy(end_, sh_end.at[pl.ds(bic * V, V)])

    vb0 = sub * VW
    if phases >= 4:
        for qi, mr in enumerate((mlx, mly, mlz, mlw)):
            for src in range(NW):
                pltpu.sync_copy(
                    sh_lap.at[pl.ds((((bic * 8 + src) * 4) + qi) * V + vb0, VW)],
                    tm.at[pl.ds(src * VW, VW)])

            def mg(i, _):
                acc = tm[pl.ds(i * 16, 16)]
                for src in range(1, NW):
                    acc = acc + tm[pl.ds(src * VW + i * 16, 16)]
                mr[pl.ds(i * 16, 16)] = acc
                return 0
            lax.fori_loop(0, VW // 16, mg, 0)

    def fin(i, acc):
        w = mlw[pl.ds(i * 16, 16)]
        safe = jnp.where(w > 0, w, 1.0)
        nw_ = jnp.where(w > 0, 1.0 / safe, 0.0)
        vxs = vx[pl.ds(vb0 + i * 16, 16)]
        vys = vy[pl.ds(vb0 + i * 16, 16)]
        vzs = vz[pl.ds(vb0 + i * 16, 16)]
        dx = mlx[pl.ds(i * 16, 16)] * nw_ - vxs
        dy = mly[pl.ds(i * 16, 16)] * nw_ - vys
        dz = mlz[pl.ds(i * 16, 16)] * nw_ - vzs
        return acc + _sqrtv(dx * dx + dy * dy + dz * dz + 1e-20)
    lap_vec = lax.fori_loop(0, VW // 16, fin, zf) if phases >= 4 else zf
    if phases >= 2:
        plsc.subcore_barrier()

    if phases >= 5:
        pltpu.sync_copy(sh_srt.at[pl.ds(bic * E, E)], srt)
        pltpu.sync_copy(sh_off.at[pl.ds(bic * V, V)], off)
        pltpu.sync_copy(sh_end.at[pl.ds(bic * V, V)], end_)
        pltpu.sync_copy(sh_eo.at[pl.ds(bic * E, E)], eo)

    def probe(i, accs):
        l2a, fca, nca, ncc = accs
        blk = i >> 5
        t0 = blk * F + fstart + (i & 31) * 16
        a_ = ea[pl.ds(t0, 16)]
        p_ = pk[pl.ds(t0, 16)]
        b_ = p_ >> 14
        t_ = t0 + iot
        j0 = plsc.load_gather(off, [a_])
        e0 = plsc.load_gather(end_, [a_])

        def cond(cst):
            j, predi, succ = cst
            return jnp.any(j < e0)

        def bdy(cst):
            j, predi, succ = cst
            jc = jnp.minimum(j, E - 1)
            px = plsc.load_gather(srt, [jc])
            bx = px >> 14
            tx = px & 16383
            eq = (bx == b_) & (j < e0)
            predi = jnp.maximum(predi, jnp.where(eq & (tx < t_), 1, 0))
            succ = jnp.minimum(succ, jnp.where(eq & (tx > t_), tx, 16384))
            return (j + 1, predi, succ)
        _, predi, succ = lax.while_loop(
            cond, bdy, (j0, zi, jnp.full((16,), 16384, jnp.int32)))

        first = predi == 0
        vxa = plsc.load_gather(vx, [a_])
        vya = plsc.load_gather(vy, [a_])
        vza = plsc.load_gather(vz, [a_])
        vxb = plsc.load_gather(vx, [b_])
        vyb = plsc.load_gather(vy, [b_])
        vzb = plsc.load_gather(vz, [b_])
        wx = vxa - vxb
        wy = vya - vyb
        wz = vza - vzb
        l2 = wx * wx + wy * wy + wz * wz
        l2a = l2a + jnp.where(first, l2, 0.0)
        fca = fca + jnp.where(first, 1.0, 0.0)

        hs = succ < 16384
        sc_ = jnp.minimum(succ, E - 1)
        o1 = eo[pl.ds(t0, 16)]
        o2 = plsc.load_gather(eo, [sc_])
        vax = plsc.load_gather(vx, [o1]) - vxb
        vay = plsc.load_gather(vy, [o1]) - vyb
        vaz = plsc.load_gather(vz, [o1]) - vzb
        vbx = plsc.load_gather(vx, [o2]) - vxb
        vby = plsc.load_gather(vy, [o2]) - vyb
        vbz = plsc.load_gather(vz, [o2]) - vzb
        n0x = vay * wz - vaz * wy
        n0y = vaz * wx - vax * wz
        n0z = vax * wy - vay * wx
        n1x = -(vby * wz - vbz * wy)
        n1y = -(vbz * wx - vbx * wz)
        n1z = -(vbx * wy - vby * wx)
        n0m = jnp.maximum(_sqrtv(n0x * n0x + n0y * n0y + n0z * n0z + 1e-20), 1e-8)
        n1m = jnp.maximum(_sqrtv(n1x * n1x + n1y * n1y + n1z * n1z + 1e-20), 1e-8)
        cosv = (n0x * n1x + n0y * n1y + n0z * n1z) / (n0m * n1m)
        nca = nca + jnp.where(hs, 1.0 - cosv, 0.0)
        ncc = ncc + jnp.where(hs, 1.0, 0.0)
        return (l2a, fca, nca, ncc)
    if phases >= 5:
        l2a, fca, nca, ncc = lax.fori_loop(0, (3 * FW) // 16, probe, (zf, zf, zf, zf))
    else:
        l2a, fca, nca, ncc = zf, zf, zf, zf

    stg[pl.ds(0, 16)] = l2a
    stg[pl.ds(16, 16)] = fca
    stg[pl.ds(32, 16)] = lap_vec
    stg[pl.ds(48, 16)] = nca
    stg[pl.ds(64, 16)] = ncc
    pltpu.sync_copy(stg, out.at[pl.ds((batch * NW + sub) * 80, 80)])


def _build_sc_kernel(phases=9):
    mesh = plsc.VectorSubcoreMesh(core_axis_name="c", subcore_axis_name="s")
    i32, f32 = jnp.int32, jnp.float32
    return functools.partial(
        pl.kernel,
        mesh=mesh,
        out_type=jax.ShapeDtypeStruct((B * NW * 80,), f32),
        scratch_types=[
            pltpu.VMEM((V,), f32), pltpu.VMEM((V,), f32), pltpu.VMEM((V,), f32),
            pltpu.VMEM((F,), i32), pltpu.VMEM((F,), i32), pltpu.VMEM((F,), i32),
            pltpu.VMEM((E,), i32), pltpu.VMEM((E,), i32), pltpu.VMEM((E,), i32),
            pltpu.VMEM((E,), i32),
            pltpu.VMEM((V,), i32), pltpu.VMEM((V,), i32), pltpu.VMEM((V,), i32),
            pltpu.VMEM((V,), f32), pltpu.VMEM((V,), f32), pltpu.VMEM((V,), f32),
            pltpu.VMEM((V,), f32),
            pltpu.VMEM((V,), f32),
            pltpu.VMEM((VW,), f32), pltpu.VMEM((VW,), f32), pltpu.VMEM((VW,), f32),
            pltpu.VMEM((VW,), f32),
            pltpu.VMEM((80,), f32),
            pltpu.VMEM_SHARED((2 * E,), i32), pltpu.VMEM_SHARED((2 * E,), i32),
            pltpu.VMEM_SHARED((2 * E,), i32), pltpu.VMEM_SHARED((2 * E,), i32),
            pltpu.VMEM_SHARED((2 * V,), i32), pltpu.VMEM_SHARED((2 * V,), i32),
            pltpu.VMEM_SHARED((2 * NW * 4 * V,), f32),
        ],
        compiler_params=pltpu.CompilerParams(needs_layout_passes=False),
    )(functools.partial(_sc_mesh_body, phases=phases))


def _mesh_losses_sc(predictions, pred_faces, phases=9):
    vflat = jnp.transpose(predictions, (0, 2, 1)).reshape(-1)
    fflat = jnp.transpose(pred_faces, (0, 2, 1)).reshape(-1).astype(jnp.int32)
    outv = _build_sc_kernel(phases)(vflat, fflat)
    pa = outv.reshape(B, NW, 5, 16).sum(axis=(1, 3))
    edge = jnp.mean(pa[:, 0] / jnp.maximum(pa[:, 1], 1.0))
    lap = jnp.mean(pa[:, 2]) / V
    nc = jnp.mean(jnp.where(pa[:, 4] > 0, pa[:, 3] / jnp.maximum(pa[:, 4], 1.0), 0.0))
    return edge, lap, nc


def _sn(x):
    return jnp.sqrt(jnp.sum(x * x, axis=1) + 1e-20)


def _edge_loss(verts, faces):
    Bn, Vn, _ = verts.shape
    F = faces.shape[1]
    total = 0.0
    for b in range(Bn):
        f = faces[b]
        e = jnp.concatenate([f[:, [0, 1]], f[:, [1, 2]], f[:, [2, 0]]], axis=0)
        e = jnp.sort(e, axis=1)
        keys = e[:, 0] * Vn + e[:, 1]
        uniq = jnp.unique(keys, size=3 * F, fill_value=-1)
        validb = uniq >= 0
        valid = validb.astype(verts.dtype)
        uq = jnp.where(validb, uniq, 0)
        v0 = verts[b][uq // Vn]
        v1 = verts[b][uq % Vn]
        l2 = jnp.sum((v0 - v1) ** 2, axis=1) * valid
        num = jnp.maximum(jnp.sum(valid), 1.0)
        total = total + jnp.sum(l2) / num
    return total / Bn


def _laplacian_loss(verts, faces):
    Bn, Vn, _ = verts.shape
    total = 0.0
    for b in range(Bn):
        vs = verts[b]
        f = faces[b]
        v0, v1, v2 = vs[f[:, 0]], vs[f[:, 1]], vs[f[:, 2]]
        A = _sn(v1 - v2)
        Bl = _sn(v0 - v2)
        C = _sn(v0 - v1)
        s = 0.5 * (A + Bl + C)
        area = jnp.sqrt(jnp.clip(s * (s - A) * (s - Bl) * (s - C), 1e-12, None))
        A2, B2, C2 = A * A, Bl * Bl, C * C
        cota = (B2 + C2 - A2) / area
        cotb = (A2 + C2 - B2) / area
        cotc = (A2 + B2 - C2) / area
        cot = (jnp.stack([cota, cotb, cotc], axis=1) / 4.0).reshape(-1)
        ii = f[:, [1, 2, 0]].reshape(-1)
        jj = f[:, [2, 0, 1]].reshape(-1)
        Lx = jnp.zeros((Vn, 3), dtype=verts.dtype)
        Lx = Lx.at[ii].add(cot[:, None] * vs[jj])
        Lx = Lx.at[jj].add(cot[:, None] * vs[ii])
        rowsum = jnp.zeros((Vn,), dtype=verts.dtype)
        rowsum = rowsum.at[ii].add(cot)
        rowsum = rowsum.at[jj].add(cot)
        safe = jnp.where(rowsum > 0, rowsum, 1.0)
        norm_w = jnp.where(rowsum > 0, 1.0 / safe, 0.0)[:, None]
        diff = Lx * norm_w - vs
        total = total + jnp.sum(_sn(diff)) / Vn
    return total / Bn


def _normal_loss(verts, faces):
    Bn, Vn, _ = verts.shape
    total = 0.0
    for b in range(Bn):
        vs = verts[b]
        f = faces[b]
        e = jnp.concatenate([f[:, [0, 1]], f[:, [1, 2]], f[:, [2, 0]]], axis=0)
        opp = jnp.concatenate([f[:, 2], f[:, 0], f[:, 1]], axis=0)
        e = jnp.sort(e, axis=1)
        keys = e[:, 0] * Vn + e[:, 1]
        order = jnp.argsort(keys)
        ks = keys[order]
        opps = opp[order]
        mask = (ks[1:] == ks[:-1]).astype(verts.dtype)
        k0 = ks[:-1]
        ev0 = vs[k0 // Vn]
        ev1 = vs[k0 % Vn]
        va = vs[opps[:-1]]
        vb = vs[opps[1:]]
        n0 = jnp.cross(va - ev1, ev0 - ev1)
        n1 = -jnp.cross(vb - ev1, ev0 - ev1)
        n0m = jnp.maximum(_sn(n0), 1e-8)
        n1m = jnp.maximum(_sn(n1), 1e-8)
        cos = jnp.sum(n0 * n1, axis=1) / (n0m * n1m)
        pair_loss = (1.0 - cos) * mask
        num = jnp.sum(mask)
        total = total + jnp.where(num > 0, jnp.sum(pair_loss) / jnp.maximum(num, 1.0), 0.0)
    return total / Bn


def kernel(predictions, targets, pred_faces):
    loss_chamfer, vel_loss = _chamfer_pallas(predictions, targets)
    edge, lap, nc = _mesh_losses_sc(predictions, pred_faces)
    return loss_chamfer + 0.05 * lap + 0.01 * nc + 0.5 * edge + 10.0 * vel_loss
